# baseline (speedup 1.0000x reference)
import functools

import jax
import jax.numpy as jnp
from jax import lax
from jax.experimental import pallas as pl
from jax.experimental.pallas import tpu as pltpu
from jax.experimental.pallas import tpu_sc as plsc

_NC = 2
_NS = 16
_NW = _NC * _NS
_LANES = 16

_RSTREAMS = 4
_QUARters = 4


@functools.lru_cache(maxsize=None)
def _make_index(B, F):
  assert F <= _NW

  @functools.partial(
      pl.kernel,
      out_type=jax.ShapeDtypeStruct((F * B,), jnp.int32),
      mesh=plsc.VectorSubcoreMesh(
          core_axis_name="c", subcore_axis_name="s",
          num_cores=_NC, num_subcores=_NS),
      scratch_types=[pltpu.VMEM((B,), jnp.int32)],
  )
  def index_kernel(xt_hbm, out_hbm, ids_v):
    wid = lax.axis_index("s") * _NC + lax.axis_index("c")

    @pl.when(wid < F)
    def _():
      pltpu.sync_copy(xt_hbm.at[wid], ids_v)
      pltpu.sync_copy(ids_v, out_hbm.at[pl.ds(wid * B, B)])

  return index_kernel


@functools.lru_cache(maxsize=None)
def _make_gather(B, F, V, D, base, count):
  per_w = count // _NW
  brows = B // 128
  qrows = brows // _QUARters
  qids = B // _QUARters
  rstep = (V // _RSTREAMS) // 128 * 128
  roffs = [r * rstep for r in range(_RSTREAMS)]
  rlens = [rstep] * (_RSTREAMS - 1) + [V - rstep * (_RSTREAMS - 1)]
  assert count % _NW == 0 and B % (128 * _QUARters) == 0

  mesh = plsc.VectorSubcoreMesh(
      core_axis_name="c", subcore_axis_name="s",
      num_cores=_NC, num_subcores=_NS)

  @functools.partial(
      pl.kernel,
      out_type=jax.ShapeDtypeStruct((count, brows, 128), jnp.float32),
      mesh=mesh,
      scratch_types=[
          pltpu.VMEM((V,), jnp.float32),
          pltpu.VMEM((qrows, 128), jnp.int32),
          pltpu.VMEM((qrows, 128), jnp.int32),
          pltpu.VMEM((brows, 128), jnp.float32),
          pltpu.SemaphoreType.DMA,
          pltpu.SemaphoreType.DMA,
          pltpu.SemaphoreType.DMA,
          pltpu.SemaphoreType.DMA,
      ],
      compiler_params=pltpu.CompilerParams(
          use_tc_tiling_on_sc=True, needs_layout_passes=False),
  )
  def gather_kernel(idx_hbm, tbl_hbm, out_hbm, row_v, ids0, ids1, out_v,
                    sem_row, sem_i0, sem_i1, sem_out):
    wid = lax.axis_index("s") * _NC + lax.axis_index("c")
    idbuf = (ids0, ids1)
    idsem = (sem_i0, sem_i1)

    @pl.loop(0, per_w)
    def _pair(i):
      o = wid * per_w + i
      p = base + o
      f = p // D
      d = p - f * D

      trow = tbl_hbm.at[f, d]
      pltpu.async_copy(trow, row_v, sem_row)
      for q in range(2):
        pltpu.async_copy(idx_hbm.at[pl.ds(f * brows + q * qrows, qrows)],
                         idbuf[q], idsem[q])

      @pl.when(i > 0)
      def _():
        pltpu.make_async_copy(out_v, out_hbm.at[o - 1], sem_out).wait()

      pltpu.make_async_copy(trow, row_v, sem_row).wait()

      for q in range(_QUARters):
        buf = idbuf[q % 2]
        pltpu.make_async_copy(
            idx_hbm.at[pl.ds(f * brows + q * qrows, qrows)], buf,
            idsem[q % 2]).wait()

        @pl.loop(0, qrows)
        def _g(rr):
          for jj in range(8):
            sl = pl.ds(jj * _LANES, _LANES)
            vals = plsc.load_gather(row_v, [buf[rr, sl]])
            out_v[q * qrows + rr, sl] = vals

        if q + 2 < _QUARters:
          pltpu.async_copy(
              idx_hbm.at[pl.ds(f * brows + (q + 2) * qrows, qrows)], buf,
              idsem[q % 2])

      pltpu.async_copy(out_v, out_hbm.at[o], sem_out)

    pltpu.make_async_copy(
        out_v, out_hbm.at[wid * per_w + per_w - 1], sem_out).wait()

  return gather_kernel


def _stats_body(x_ref, o_ref):
  @pl.when(pl.program_id(0) == 0)
  def _():
    o_ref[...] = jnp.zeros_like(o_ref)

  xb = x_ref[...]
  s = jnp.sum(jnp.sum(xb, axis=1), axis=1, keepdims=True)
  sq = jnp.sum(jnp.sum(xb * xb, axis=1), axis=1, keepdims=True)
  o_ref[:, 0:1] += s
  o_ref[:, 1:2] += sq


def _head_body(nb_inv, rb, xa_ref, xb_ref, sta_ref, stb_ref, g_ref, be_ref,
               w_ref, b_ref, o_ref):
  st = jnp.concatenate([sta_ref[...], stb_ref[...]], axis=0)
  mean = st[:, 0:1] * nb_inv
  var = st[:, 1:2] * nb_inv - mean * mean
  scale = g_ref[...] * lax.rsqrt(var + 1e-5)
  shift = be_ref[...] - mean * scale
  ka = xa_ref.shape[0]
  kb = xb_ref.shape[0]
  xm = jnp.concatenate([xa_ref[...].reshape(ka, rb * 128),
                        xb_ref[...].reshape(kb, rb * 128)], axis=0)
  xn = xm * scale + shift
  yt = lax.dot_general(w_ref[...], xn, (((1,), (0,)), ((), ())),
                       preferred_element_type=jnp.float32)
  y = yt.T
  o_ref[...] = jnp.maximum(y + b_ref[...], 0.0)


@functools.lru_cache(maxsize=None)
def _make_stats(B, k, rb):
  nb = B // 128 // rb
  return pl.pallas_call(
      _stats_body,
      grid=(nb,),
      in_specs=[pl.BlockSpec((k, rb, 128), lambda i: (0, i, 0))],
      out_specs=pl.BlockSpec((k, 2), lambda i: (0, 0)),
      out_shape=jax.ShapeDtypeStruct((k, 2), jnp.float32),
  )


@functools.lru_cache(maxsize=None)
def _make_head(B, ka, kb, OUT, rb):
  K = ka + kb
  nb = B // 128 // rb
  return pl.pallas_call(
      functools.partial(_head_body, 1.0 / B, rb),
      grid=(nb,),
      in_specs=[
          pl.BlockSpec((ka, rb, 128), lambda i: (0, i, 0)),
          pl.BlockSpec((kb, rb, 128), lambda i: (0, i, 0)),
          pl.BlockSpec((ka, 2), lambda i: (0, 0)),
          pl.BlockSpec((kb, 2), lambda i: (0, 0)),
          pl.BlockSpec((K, 1), lambda i: (0, 0)),
          pl.BlockSpec((K, 1), lambda i: (0, 0)),
          pl.BlockSpec((OUT, K), lambda i: (0, 0)),
          pl.BlockSpec((1, OUT), lambda i: (0, 0)),
      ],
      out_specs=pl.BlockSpec((rb * 128, OUT), lambda i: (i, 0)),
      out_shape=jax.ShapeDtypeStruct((B, OUT), jnp.float32),
  )


def kernel(x_in, emb_tables, bn_gamma, bn_beta, W, b):
  B, F = x_in.shape
  _, V, D = emb_tables.shape
  OUT = W.shape[0]
  K = F * D

  xt = x_in.T
  tblT = emb_tables.transpose(0, 2, 1)

  ids = _make_index(B, F)(xt)
  idx2 = ids.reshape(F * B // 128, 128)
  ka = 224
  kb = K - ka
  x3a = _make_gather(B, F, V, D, 0, ka)(idx2, tblT)
  x3b = _make_gather(B, F, V, D, ka, kb)(idx2, tblT)
  sta = _make_stats(B, ka, 16)(x3a)
  stb = _make_stats(B, kb, 16)(x3b)
  return _make_head(B, ka, kb, OUT, 16)(
      x3a, x3b, sta, stb, bn_gamma.reshape(K, 1), bn_beta.reshape(K, 1),
      W, b.reshape(1, OUT))


# speedup vs baseline: 1.0344x; 1.0344x over previous
import functools

import jax
import jax.numpy as jnp
from jax import lax
from jax.experimental import pallas as pl
from jax.experimental.pallas import tpu as pltpu
from jax.experimental.pallas import tpu_sc as plsc

_NC = 2
_NS = 16
_NW = _NC * _NS
_LANES = 16

_RSTREAMS = 4
_QUARters = 4


@functools.lru_cache(maxsize=None)
def _make_index(B, F):
  assert F <= _NW

  @functools.partial(
      pl.kernel,
      out_type=jax.ShapeDtypeStruct((F * B,), jnp.int32),
      mesh=plsc.VectorSubcoreMesh(
          core_axis_name="c", subcore_axis_name="s",
          num_cores=_NC, num_subcores=_NS),
      scratch_types=[pltpu.VMEM((B,), jnp.int32)],
  )
  def index_kernel(xt_hbm, out_hbm, ids_v):
    wid = lax.axis_index("s") * _NC + lax.axis_index("c")

    @pl.when(wid < F)
    def _():
      pltpu.sync_copy(xt_hbm.at[wid], ids_v)
      pltpu.sync_copy(ids_v, out_hbm.at[pl.ds(wid * B, B)])

  return index_kernel


@functools.lru_cache(maxsize=None)
def _make_gather(B, F, V, D, base, count):
  per_w = count // _NW
  brows = B // 128
  qrows = brows // _QUARters
  qids = B // _QUARters
  rstep = (V // _RSTREAMS) // 128 * 128
  roffs = [r * rstep for r in range(_RSTREAMS)]
  rlens = [rstep] * (_RSTREAMS - 1) + [V - rstep * (_RSTREAMS - 1)]
  assert count % _NW == 0 and B % (128 * _QUARters) == 0

  mesh = plsc.VectorSubcoreMesh(
      core_axis_name="c", subcore_axis_name="s",
      num_cores=_NC, num_subcores=_NS)

  @functools.partial(
      pl.kernel,
      out_type=jax.ShapeDtypeStruct((count, brows, 128), jnp.float32),
      mesh=mesh,
      scratch_types=[
          pltpu.VMEM((V,), jnp.float32),
          pltpu.VMEM((qrows, 128), jnp.int32),
          pltpu.VMEM((qrows, 128), jnp.int32),
          pltpu.VMEM((brows, 128), jnp.float32),
          pltpu.SemaphoreType.DMA,
          pltpu.SemaphoreType.DMA,
          pltpu.SemaphoreType.DMA,
          pltpu.SemaphoreType.DMA,
      ],
      compiler_params=pltpu.CompilerParams(
          use_tc_tiling_on_sc=True, needs_layout_passes=False),
  )
  def gather_kernel(idx_hbm, tbl_hbm, out_hbm, row_v, ids0, ids1, out_v,
                    sem_row, sem_i0, sem_i1, sem_out):
    wid = lax.axis_index("s") * _NC + lax.axis_index("c")
    idbuf = (ids0, ids1)
    idsem = (sem_i0, sem_i1)

    @pl.loop(0, per_w)
    def _pair(i):
      o = wid * per_w + i
      p = base + o
      f = p // D
      d = p - f * D

      trow = tbl_hbm.at[f, d]
      pltpu.async_copy(trow, row_v, sem_row)
      for q in range(2):
        pltpu.async_copy(idx_hbm.at[pl.ds(f * brows + q * qrows, qrows)],
                         idbuf[q], idsem[q])

      @pl.when(i > 0)
      def _():
        pltpu.make_async_copy(out_v, out_hbm.at[o - 1], sem_out).wait()

      pltpu.make_async_copy(trow, row_v, sem_row).wait()

      for q in range(_QUARters):
        buf = idbuf[q % 2]
        pltpu.make_async_copy(
            idx_hbm.at[pl.ds(f * brows + q * qrows, qrows)], buf,
            idsem[q % 2]).wait()

        @pl.loop(0, qrows)
        def _g(rr):
          for jj in range(8):
            sl = pl.ds(jj * _LANES, _LANES)
            vals = plsc.load_gather(row_v, [buf[rr, sl]])
            out_v[q * qrows + rr, sl] = vals

        if q + 2 < _QUARters:
          pltpu.async_copy(
              idx_hbm.at[pl.ds(f * brows + (q + 2) * qrows, qrows)], buf,
              idsem[q % 2])

      pltpu.async_copy(out_v, out_hbm.at[o], sem_out)

    pltpu.make_async_copy(
        out_v, out_hbm.at[wid * per_w + per_w - 1], sem_out).wait()

  return gather_kernel


def _stats_body(x_ref, o_ref):
  @pl.when(pl.program_id(0) == 0)
  def _():
    o_ref[...] = jnp.zeros_like(o_ref)

  xb = x_ref[...]
  s = jnp.sum(jnp.sum(xb, axis=1), axis=1, keepdims=True)
  sq = jnp.sum(jnp.sum(xb * xb, axis=1), axis=1, keepdims=True)
  o_ref[:, 0:1] += s
  o_ref[:, 1:2] += sq


def _head_body(nb_inv, rb, x_ref, st_ref, g_ref, be_ref, w_ref, b_ref,
               o_ref):
  st = st_ref[...]
  mean = st[:, 0:1] * nb_inv
  var = st[:, 1:2] * nb_inv - mean * mean
  scale = g_ref[...] * lax.rsqrt(var + 1e-5)
  shift = be_ref[...] - mean * scale
  k = x_ref.shape[0]
  xn = x_ref[...].reshape(k, rb * 128) * scale + shift
  yt = lax.dot_general(w_ref[...], xn, (((1,), (0,)), ((), ())),
                       preferred_element_type=jnp.float32)
  y = yt.T
  o_ref[...] = jnp.maximum(y + b_ref[...], 0.0)


@functools.lru_cache(maxsize=None)
def _make_stats(B, k, rb):
  nb = B // 128 // rb
  return pl.pallas_call(
      _stats_body,
      grid=(nb,),
      in_specs=[pl.BlockSpec((k, rb, 128), lambda i: (0, i, 0))],
      out_specs=pl.BlockSpec((k, 2), lambda i: (0, 0)),
      out_shape=jax.ShapeDtypeStruct((k, 2), jnp.float32),
  )


@functools.lru_cache(maxsize=None)
def _make_head(B, K, OUT, rb):
  nb = B // 128 // rb
  return pl.pallas_call(
      functools.partial(_head_body, 1.0 / B, rb),
      grid=(nb,),
      in_specs=[
          pl.BlockSpec((K, rb, 128), lambda i: (0, i, 0)),
          pl.BlockSpec((K, 2), lambda i: (0, 0)),
          pl.BlockSpec((K, 1), lambda i: (0, 0)),
          pl.BlockSpec((K, 1), lambda i: (0, 0)),
          pl.BlockSpec((OUT, K), lambda i: (0, 0)),
          pl.BlockSpec((1, OUT), lambda i: (0, 0)),
      ],
      out_specs=pl.BlockSpec((rb * 128, OUT), lambda i: (i, 0)),
      out_shape=jax.ShapeDtypeStruct((B, OUT), jnp.float32),
  )


def kernel(x_in, emb_tables, bn_gamma, bn_beta, W, b):
  B, F = x_in.shape
  _, V, D = emb_tables.shape
  OUT = W.shape[0]
  K = F * D

  xt = x_in.T
  tblT = emb_tables.transpose(0, 2, 1)

  ids = _make_index(B, F)(xt)
  idx2 = ids.reshape(F * B // 128, 128)
  x3 = _make_gather(B, F, V, D, 0, K)(idx2, tblT)
  st = _make_stats(B, K, 16)(x3)
  return _make_head(B, K, OUT, 16)(
      x3, st, bn_gamma.reshape(K, 1), bn_beta.reshape(K, 1),
      W, b.reshape(1, OUT))
